# baseline (speedup 1.0000x reference)
import functools

import jax
import jax.numpy as jnp
from jax import lax
from jax.experimental import pallas as pl
from jax.experimental.pallas import tpu as pltpu
from jax.experimental.pallas import tpu_sc as plsc

N = 10000
E = 160000
NG = 128
NP = 10240
ET = E // 16
FB = ET // 128
TAIL = ET - FB * 128
NPT = NP // 16
GH = NG // 2
DEAD = NP * GH
SW = NP * GH + 128
GPAD = 200

_MESH = plsc.VectorSubcoreMesh(core_axis_name="c", subcore_axis_name="s")


def _zero_fill(ref, nwords):
    z = jnp.zeros((16,), ref.dtype)

    def body(i, carry):
        ref[pl.ds(i * 16, 16)] = z
        return carry

    lax.fori_loop(0, nwords // 16, body, 0)


def _newton_rsqrt(d):
    i = plsc.bitcast(d, jnp.int32)
    i = 0x5F3759DF - lax.shift_right_logical(i, 1)
    y = plsc.bitcast(i, jnp.float32)
    for _ in range(3):
        y = y * (1.5 - 0.5 * d * y * y)
    return y


@functools.partial(
    pl.kernel,
    out_type=(
        jax.ShapeDtypeStruct((2, NP * GH), jnp.float32),
        jax.ShapeDtypeStruct((2, 256), jnp.float32),
    ),
    mesh=_MESH,
    compiler_params=pltpu.CompilerParams(needs_layout_passes=False),
    scratch_types=[
        pltpu.VMEM((ET,), jnp.int32),
        pltpu.VMEM((ET,), jnp.int32),
        pltpu.VMEM((ET,), jnp.float32),
        pltpu.VMEM((NP,), jnp.int32),
        pltpu.VMEM((NP,), jnp.float32),
        pltpu.VMEM((NP,), jnp.float32),
        pltpu.VMEM((NPT,), jnp.float32),
        pltpu.VMEM((NPT,), jnp.float32),
        pltpu.VMEM((NPT,), jnp.float32),
        pltpu.VMEM((128,), jnp.int32),
        pltpu.VMEM((128,), jnp.int32),
        pltpu.VMEM((128,), jnp.int32),
        pltpu.VMEM((128,), jnp.int32),
        pltpu.VMEM((128,), jnp.float32),
        pltpu.VMEM((128,), jnp.float32),
        pltpu.VMEM((128,), jnp.float32),
        pltpu.VMEM((128,), jnp.float32),
        pltpu.VMEM((16,), jnp.int32),
        pltpu.VMEM((16,), jnp.float32),
        pltpu.VMEM((8192,), jnp.float32),
        pltpu.VMEM((256,), jnp.float32),
        pltpu.VMEM((16, 256), jnp.float32),
        pltpu.VMEM_SHARED((NP,), jnp.float32),
        pltpu.VMEM_SHARED((16, 256), jnp.float32),
        pltpu.VMEM_SHARED((NP,), jnp.float32),
        pltpu.VMEM_SHARED((NP,), jnp.float32),
        pltpu.VMEM_SHARED((SW,), jnp.float32),
        pltpu.SemaphoreType.DMA,
        pltpu.SemaphoreType.DMA,
        pltpu.SemaphoreType.DMA,
        pltpu.SemaphoreType.DMA,
        pltpu.SemaphoreType.DMA,
        pltpu.SemaphoreType.DMA,
        pltpu.SemaphoreType.DMA,
        pltpu.SemaphoreType.DMA,
    ],
)
def _gcn_sc(eif, ew, bat, sp, cnt,
            srcv, dstv, ewv, batv, disv, invv, degv, disl, invl,
            idx0, idx1, idx2, idx3, val0, val1, val2, val3, idxt, valt, zb,
            cntloc, redbuf, degsh, cntsl, dissh, invsh, ssh,
            sem0, sem1, sem2, sem3, sem4, sem5, sem6, sem7):
    c = lax.axis_index("c")
    s = lax.axis_index("s")
    glo = c * GH
    lanes = lax.iota(jnp.int32, 16)
    slots = ((idx0, val0, sem0), (idx1, val1, sem1),
             (idx2, val2, sem2), (idx3, val3, sem3))

    pltpu.async_copy(eif.at[pl.ds(s * ET, ET)], srcv, sem4)
    pltpu.async_copy(eif.at[pl.ds(E + s * ET, ET)], dstv, sem5)
    pltpu.async_copy(ew.at[pl.ds(s * ET, ET)], ewv, sem6)
    pltpu.async_copy(bat, batv, sem7)

    _zero_fill(zb, 8192)

    ts = NP * GH // 16

    for i in range(4):
        pltpu.async_copy(zb, ssh.at[pl.ds(s * ts + i * 8192, 8192)],
                         (sem0, sem1, sem2, sem3)[i])
    _zero_fill(cntloc, 256)
    pltpu.sync_copy(zb.at[pl.ds(0, NPT)], degsh.at[pl.ds(s * NPT, NPT)])

    pltpu.make_async_copy(zb, ssh.at[pl.ds(s * ts, 8192)], sem0).wait()
    pltpu.sync_copy(zb, ssh.at[pl.ds(s * ts + 4 * 8192, 8192)])
    for i in range(1, 4):
        pltpu.make_async_copy(zb, ssh.at[pl.ds(s * ts + i * 8192, 8192)],
                              (sem0, sem1, sem2, sem3)[i]).wait()
    pltpu.make_async_copy(eif.at[pl.ds(s * ET, ET)], srcv, sem4).wait()
    pltpu.make_async_copy(eif.at[pl.ds(E + s * ET, ET)], dstv, sem5).wait()
    pltpu.make_async_copy(ew.at[pl.ds(s * ET, ET)], ewv, sem6).wait()
    pltpu.make_async_copy(bat, batv, sem7).wait()

    def _pipeline(nbatches, make_batch, target):
        nslots = len(slots)

        def fire(j, sl):
            idxr, valr, sem = slots[sl]
            make_batch(j, idxr, valr)
            pltpu.async_copy(valr, target.at[idxr], sem, add=True)

        def drain(sl):
            idxr, valr, sem = slots[sl]
            pltpu.make_async_copy(valr, target.at[idxr], sem).wait()

        for b in range(nslots):
            fire(b, b)

        def grp(g, carry):
            for b in range(nslots):
                drain(b)
                fire(nslots + g * nslots + b, b)
            return carry

        ngroups = (nbatches - nslots) // nslots
        lax.fori_loop(0, ngroups, grp, 0)
        rem = (nbatches - nslots) % nslots
        for r in range(rem):
            drain(r)
            fire(nbatches - rem + r, r)
        for b in range(nslots):
            drain(b)

    def deg_batch(j, idxr, valr):
        def sub(k, carry):
            off = j * 128 + k * 16
            idxr[pl.ds(k * 16, 16)] = dstv[pl.ds(off, 16)]
            valr[pl.ds(k * 16, 16)] = ewv[pl.ds(off, 16)]
            return carry

        lax.fori_loop(0, 8, sub, 0)

    _pipeline(FB, deg_batch, degsh)

    idxt[...] = dstv[pl.ds(FB * 128, 16)]
    valt[...] = ewv[pl.ds(FB * 128, 16)]
    pltpu.sync_copy(valt, degsh.at[idxt], add=True)

    def cacc(i, carry):
        base = s * NPT + i * 16
        b16 = batv[pl.ds(base, 16)]
        valid = base + lanes < N
        plsc.addupdate_scatter(cntloc,
                               [jnp.where(valid, b16, GPAD + lanes)],
                               jnp.where(valid, 1.0, 0.0))
        return carry

    lax.fori_loop(0, NPT // 16, cacc, 0)

    pltpu.sync_copy(cntloc, cntsl.at[s])
    plsc.subcore_barrier()

    pltpu.sync_copy(cntsl, redbuf)

    def cred(i, carry):
        acc = redbuf[0, pl.ds(i * 16, 16)]
        for r in range(1, 16):
            acc = acc + redbuf[r, pl.ds(i * 16, 16)]
        cntloc[pl.ds(i * 16, 16)] = acc
        return carry

    lax.fori_loop(0, 256 // 16, cred, 0)

    pltpu.sync_copy(degsh.at[pl.ds(s * NPT, NPT)], degv)

    def newton(i, carry):
        d = degv[pl.ds(i * 16, 16)] + 1.0
        y = _newton_rsqrt(d)
        disl[pl.ds(i * 16, 16)] = y
        invl[pl.ds(i * 16, 16)] = y * y
        return carry

    lax.fori_loop(0, NPT // 16, newton, 0)
    pltpu.sync_copy(disl, dissh.at[pl.ds(s * NPT, NPT)])
    pltpu.sync_copy(invl, invsh.at[pl.ds(s * NPT, NPT)])
    plsc.subcore_barrier()
    pltpu.sync_copy(dissh, disv)
    pltpu.sync_copy(invsh, invv)

    def s_batch(j, idxr, valr):
        def sub(k, carry):
            off = j * 128 + k * 16
            s16 = srcv[pl.ds(off, 16)]
            d16 = dstv[pl.ds(off, 16)]
            w16 = ewv[pl.ds(off, 16)]
            dd = plsc.load_gather(disv, [d16])
            ds_ = plsc.load_gather(disv, [s16])
            gi = plsc.load_gather(batv, [d16]) - glo
            own = (gi >= 0) & (gi < GH)
            dead = DEAD + ((lanes + k * 16) & 127)
            idxr[pl.ds(k * 16, 16)] = jnp.where(own, gi * NP + s16, dead)
            valr[pl.ds(k * 16, 16)] = jnp.where(own, dd * ds_ * w16, 0.0)
            return carry

        lax.fori_loop(0, 8, sub, 0)

    _pipeline(FB, s_batch, ssh)

    s16 = srcv[pl.ds(FB * 128, 16)]
    d16 = dstv[pl.ds(FB * 128, 16)]
    w16 = ewv[pl.ds(FB * 128, 16)]
    dd = plsc.load_gather(disv, [d16])
    ds_ = plsc.load_gather(disv, [s16])
    gi = plsc.load_gather(batv, [d16]) - glo
    own = (gi >= 0) & (gi < GH)
    idxt[...] = jnp.where(own, gi * NP + s16, DEAD + lanes)
    valt[...] = jnp.where(own, dd * ds_ * w16, 0.0)
    pltpu.sync_copy(valt, ssh.at[idxt], add=True)

    def self_batch(j, idxr, valr):
        def sub(k, carry):
            base = j * 128 + k * 16
            n16 = s * NPT + base + lanes
            gi = plsc.load_gather(batv, [n16]) - glo
            v16 = plsc.load_gather(invv, [n16])
            own = (gi >= 0) & (gi < GH) & (n16 < N)
            dead = DEAD + ((lanes + k * 16) & 127)
            idxr[pl.ds(k * 16, 16)] = jnp.where(own, gi * NP + n16, dead)
            valr[pl.ds(k * 16, 16)] = jnp.where(own, v16, 0.0)
            return carry

        lax.fori_loop(0, 8, sub, 0)

    _pipeline(NPT // 128, self_batch, ssh)

    plsc.subcore_barrier()

    pltpu.sync_copy(ssh.at[pl.ds(s * ts, ts)], sp.at[c, pl.ds(s * ts, ts)])

    @pl.when(s == 0)
    def _():
        pltpu.sync_copy(cntloc, cnt.at[c])


_BN = 1024


def _h_tc_body(x_ref, w_ref, h_ref):
    k = pl.program_id(0)
    h = jnp.dot(x_ref[...], w_ref[...], preferred_element_type=jnp.float32)
    rid = lax.broadcasted_iota(jnp.int32, h.shape, 0)
    h_ref[...] = jnp.where(rid < N - k * _BN, h, 0.0)


def _p_tc_body(s_ref, h_ref, cnt_ref, b_ref, out_ref, acc_ref):
    k = pl.program_id(0)

    @pl.when(k == 0)
    def _():
        acc_ref[...] = jnp.zeros_like(acc_ref)

    h = h_ref[...]
    acc_ref[0:GH, :] += jnp.dot(s_ref[0], h,
                                preferred_element_type=jnp.float32)
    acc_ref[GH:NG, :] += jnp.dot(s_ref[1], h,
                                 preferred_element_type=jnp.float32)

    @pl.when(k == pl.num_programs(0) - 1)
    def _():
        cnt = cnt_ref[...]
        pooled = (acc_ref[...] + cnt * b_ref[...]) / jnp.maximum(cnt, 1.0)
        m = jnp.max(pooled, axis=1, keepdims=True)
        shifted = pooled - m
        out_ref[...] = shifted - jnp.log(
            jnp.sum(jnp.exp(shifted), axis=1, keepdims=True))


def kernel(x, edge_index, edge_weights, batch, W, b):
    n, d = x.shape
    cdim = W.shape[1]

    batp = jnp.concatenate([batch, jnp.zeros((NP - n,), jnp.int32)])
    eif = edge_index.reshape(2 * E)
    sp, cnt = _gcn_sc(eif, edge_weights, batp)
    s3 = sp.reshape(2, GH, NP)
    cnt_col = cnt[0, :NG].reshape(NG, 1)
    b_row = b.reshape(1, cdim)

    h = pl.pallas_call(
        _h_tc_body,
        grid=(NP // _BN,),
        in_specs=[
            pl.BlockSpec((_BN, d), lambda k: (k, 0)),
            pl.BlockSpec((d, cdim), lambda k: (0, 0)),
        ],
        out_specs=pl.BlockSpec((_BN, cdim), lambda k: (k, 0)),
        out_shape=jax.ShapeDtypeStruct((NP, cdim), jnp.float32),
    )(x, W)

    out = pl.pallas_call(
        _p_tc_body,
        grid=(NP // _BN,),
        in_specs=[
            pl.BlockSpec((2, GH, _BN), lambda k: (0, 0, k)),
            pl.BlockSpec((_BN, cdim), lambda k: (k, 0)),
            pl.BlockSpec((NG, 1), lambda k: (0, 0)),
            pl.BlockSpec((1, cdim), lambda k: (0, 0)),
        ],
        out_specs=pl.BlockSpec((NG, cdim), lambda k: (0, 0)),
        out_shape=jax.ShapeDtypeStruct((NG, cdim), jnp.float32),
        scratch_shapes=[pltpu.VMEM((NG, cdim), jnp.float32)],
    )(s3, h, cnt_col, b_row)

    return out


# speedup vs baseline: 1.0130x; 1.0130x over previous
import functools

import jax
import jax.numpy as jnp
from jax import lax
from jax.experimental import pallas as pl
from jax.experimental.pallas import tpu as pltpu
from jax.experimental.pallas import tpu_sc as plsc

N = 10000
E = 160000
NG = 128
NP = 10240
ET = E // 16
FB = ET // 128
TAIL = ET - FB * 128
NPT = NP // 16
GH = NG // 2
DEAD = NP * GH
SW = NP * GH + 128
GPAD = 200

_MESH = plsc.VectorSubcoreMesh(core_axis_name="c", subcore_axis_name="s")


def _zero_fill(ref, nwords):
    z = jnp.zeros((16,), ref.dtype)

    def body(i, carry):
        ref[pl.ds(i * 16, 16)] = z
        return carry

    lax.fori_loop(0, nwords // 16, body, 0)


def _newton_rsqrt(d):
    i = plsc.bitcast(d, jnp.int32)
    i = 0x5F3759DF - lax.shift_right_logical(i, 1)
    y = plsc.bitcast(i, jnp.float32)
    for _ in range(3):
        y = y * (1.5 - 0.5 * d * y * y)
    return y


@functools.partial(
    pl.kernel,
    out_type=(
        jax.ShapeDtypeStruct((2, NP * GH), jnp.float32),
        jax.ShapeDtypeStruct((2, 256), jnp.float32),
    ),
    mesh=_MESH,
    compiler_params=pltpu.CompilerParams(needs_layout_passes=False),
    scratch_types=[
        pltpu.VMEM((ET,), jnp.int32),
        pltpu.VMEM((ET,), jnp.int32),
        pltpu.VMEM((ET,), jnp.float32),
        pltpu.VMEM((NP,), jnp.int32),
        pltpu.VMEM((NP,), jnp.float32),
        pltpu.VMEM((NP,), jnp.float32),
        pltpu.VMEM((NPT,), jnp.float32),
        pltpu.VMEM((NPT,), jnp.float32),
        pltpu.VMEM((NPT,), jnp.float32),
        pltpu.VMEM((128,), jnp.int32),
        pltpu.VMEM((128,), jnp.int32),
        pltpu.VMEM((128,), jnp.int32),
        pltpu.VMEM((128,), jnp.int32),
        pltpu.VMEM((128,), jnp.float32),
        pltpu.VMEM((128,), jnp.float32),
        pltpu.VMEM((128,), jnp.float32),
        pltpu.VMEM((128,), jnp.float32),
        pltpu.VMEM((16,), jnp.int32),
        pltpu.VMEM((16,), jnp.float32),
        pltpu.VMEM((8192,), jnp.float32),
        pltpu.VMEM_SHARED((NP,), jnp.float32),
        pltpu.VMEM_SHARED((256,), jnp.float32),
        pltpu.VMEM_SHARED((NP,), jnp.float32),
        pltpu.VMEM_SHARED((NP,), jnp.float32),
        pltpu.VMEM_SHARED((SW,), jnp.float32),
        pltpu.SemaphoreType.DMA,
        pltpu.SemaphoreType.DMA,
        pltpu.SemaphoreType.DMA,
        pltpu.SemaphoreType.DMA,
        pltpu.SemaphoreType.DMA,
        pltpu.SemaphoreType.DMA,
        pltpu.SemaphoreType.DMA,
        pltpu.SemaphoreType.DMA,
    ],
)
def _gcn_sc(eif, ew, bat, sp, cnt,
            srcv, dstv, ewv, batv, disv, invv, degv, disl, invl,
            idx0, idx1, idx2, idx3, val0, val1, val2, val3, idxt, valt, zb,
            degsh, cntsh, dissh, invsh, ssh,
            sem0, sem1, sem2, sem3, sem4, sem5, sem6, sem7):
    c = lax.axis_index("c")
    s = lax.axis_index("s")
    glo = c * GH
    lanes = lax.iota(jnp.int32, 16)
    slots = ((idx0, val0, sem0), (idx1, val1, sem1),
             (idx2, val2, sem2), (idx3, val3, sem3))

    pltpu.async_copy(eif.at[pl.ds(s * ET, ET)], srcv, sem4)
    pltpu.async_copy(eif.at[pl.ds(E + s * ET, ET)], dstv, sem5)
    pltpu.async_copy(ew.at[pl.ds(s * ET, ET)], ewv, sem6)
    pltpu.async_copy(bat, batv, sem7)

    _zero_fill(zb, 8192)

    ts = NP * GH // 16

    for i in range(4):
        pltpu.async_copy(zb, ssh.at[pl.ds(s * ts + i * 8192, 8192)],
                         (sem0, sem1, sem2, sem3)[i])
    pltpu.sync_copy(zb.at[pl.ds(0, NPT)], degsh.at[pl.ds(s * NPT, NPT)])

    @pl.when(s == 0)
    def _():
        pltpu.sync_copy(zb.at[pl.ds(0, 256)], cntsh)

    pltpu.make_async_copy(zb, ssh.at[pl.ds(s * ts, 8192)], sem0).wait()
    pltpu.sync_copy(zb, ssh.at[pl.ds(s * ts + 4 * 8192, 8192)])
    for i in range(1, 4):
        pltpu.make_async_copy(zb, ssh.at[pl.ds(s * ts + i * 8192, 8192)],
                              (sem0, sem1, sem2, sem3)[i]).wait()
    pltpu.make_async_copy(eif.at[pl.ds(s * ET, ET)], srcv, sem4).wait()
    pltpu.make_async_copy(eif.at[pl.ds(E + s * ET, ET)], dstv, sem5).wait()
    pltpu.make_async_copy(ew.at[pl.ds(s * ET, ET)], ewv, sem6).wait()
    pltpu.make_async_copy(bat, batv, sem7).wait()

    def _pipeline(nbatches, make_batch, target):
        nslots = len(slots)

        def fire(j, sl):
            idxr, valr, sem = slots[sl]
            make_batch(j, idxr, valr)
            pltpu.async_copy(valr, target.at[idxr], sem, add=True)

        def drain(sl):
            idxr, valr, sem = slots[sl]
            pltpu.make_async_copy(valr, target.at[idxr], sem).wait()

        for b in range(nslots):
            fire(b, b)

        def grp(g, carry):
            for b in range(nslots):
                drain(b)
                fire(nslots + g * nslots + b, b)
            return carry

        ngroups = (nbatches - nslots) // nslots
        lax.fori_loop(0, ngroups, grp, 0)
        rem = (nbatches - nslots) % nslots
        for r in range(rem):
            drain(r)
            fire(nbatches - rem + r, r)
        for b in range(nslots):
            drain(b)

    def deg_batch(j, idxr, valr):
        def sub(k, carry):
            off = j * 128 + k * 16
            idxr[pl.ds(k * 16, 16)] = dstv[pl.ds(off, 16)]
            valr[pl.ds(k * 16, 16)] = ewv[pl.ds(off, 16)]
            return carry

        lax.fori_loop(0, 8, sub, 0)

    _pipeline(FB, deg_batch, degsh)

    idxt[...] = dstv[pl.ds(FB * 128, 16)]
    valt[...] = ewv[pl.ds(FB * 128, 16)]
    pltpu.sync_copy(valt, degsh.at[idxt], add=True)

    def count_batch(j, idxr, valr):
        def sub(k, carry):
            base = j * 128 + k * 16
            nglob = s * NPT + base + lanes
            b16 = batv[pl.ds(s * NPT + base, 16)]
            valid = nglob < N
            idxr[pl.ds(k * 16, 16)] = jnp.where(valid, b16, GPAD + lanes)
            valr[pl.ds(k * 16, 16)] = jnp.where(valid, 1.0, 0.0)
            return carry

        lax.fori_loop(0, 8, sub, 0)

    _pipeline(NPT // 128, count_batch, cntsh)

    plsc.subcore_barrier()

    pltpu.sync_copy(degsh.at[pl.ds(s * NPT, NPT)], degv)

    def newton(i, carry):
        d = degv[pl.ds(i * 16, 16)] + 1.0
        y = _newton_rsqrt(d)
        disl[pl.ds(i * 16, 16)] = y
        invl[pl.ds(i * 16, 16)] = y * y
        return carry

    lax.fori_loop(0, NPT // 16, newton, 0)
    pltpu.sync_copy(disl, dissh.at[pl.ds(s * NPT, NPT)])
    pltpu.sync_copy(invl, invsh.at[pl.ds(s * NPT, NPT)])
    plsc.subcore_barrier()
    pltpu.sync_copy(dissh, disv)
    pltpu.sync_copy(invsh, invv)

    def s_batch(j, idxr, valr):
        def sub(k, carry):
            off = j * 128 + k * 16
            s16 = srcv[pl.ds(off, 16)]
            d16 = dstv[pl.ds(off, 16)]
            w16 = ewv[pl.ds(off, 16)]
            dd = plsc.load_gather(disv, [d16])
            ds_ = plsc.load_gather(disv, [s16])
            gi = plsc.load_gather(batv, [d16]) - glo
            own = (gi >= 0) & (gi < GH)
            dead = DEAD + ((lanes + k * 16) & 127)
            idxr[pl.ds(k * 16, 16)] = jnp.where(own, gi * NP + s16, dead)
            valr[pl.ds(k * 16, 16)] = jnp.where(own, dd * ds_ * w16, 0.0)
            return carry

        lax.fori_loop(0, 8, sub, 0)

    _pipeline(FB, s_batch, ssh)

    s16 = srcv[pl.ds(FB * 128, 16)]
    d16 = dstv[pl.ds(FB * 128, 16)]
    w16 = ewv[pl.ds(FB * 128, 16)]
    dd = plsc.load_gather(disv, [d16])
    ds_ = plsc.load_gather(disv, [s16])
    gi = plsc.load_gather(batv, [d16]) - glo
    own = (gi >= 0) & (gi < GH)
    idxt[...] = jnp.where(own, gi * NP + s16, DEAD + lanes)
    valt[...] = jnp.where(own, dd * ds_ * w16, 0.0)
    pltpu.sync_copy(valt, ssh.at[idxt], add=True)

    def self_batch(j, idxr, valr):
        def sub(k, carry):
            base = j * 128 + k * 16
            n16 = s * NPT + base + lanes
            gi = plsc.load_gather(batv, [n16]) - glo
            v16 = plsc.load_gather(invv, [n16])
            own = (gi >= 0) & (gi < GH) & (n16 < N)
            dead = DEAD + ((lanes + k * 16) & 127)
            idxr[pl.ds(k * 16, 16)] = jnp.where(own, gi * NP + n16, dead)
            valr[pl.ds(k * 16, 16)] = jnp.where(own, v16, 0.0)
            return carry

        lax.fori_loop(0, 8, sub, 0)

    _pipeline(NPT // 128, self_batch, ssh)

    plsc.subcore_barrier()

    pltpu.sync_copy(ssh.at[pl.ds(s * ts, ts)], sp.at[c, pl.ds(s * ts, ts)])

    @pl.when(s == 0)
    def _():
        pltpu.sync_copy(cntsh, cnt.at[c])


_BN = 1024


def _h_tc_body(x_ref, w_ref, h_ref):
    k = pl.program_id(0)
    h = jnp.dot(x_ref[...], w_ref[...], preferred_element_type=jnp.float32)
    rid = lax.broadcasted_iota(jnp.int32, h.shape, 0)
    h_ref[...] = jnp.where(rid < N - k * _BN, h, 0.0)


def _p_tc_body(s_ref, h_ref, cnt_ref, b_ref, out_ref, acc_ref):
    k = pl.program_id(0)

    @pl.when(k == 0)
    def _():
        acc_ref[...] = jnp.zeros_like(acc_ref)

    h = h_ref[...]
    acc_ref[0:GH, :] += jnp.dot(s_ref[0], h,
                                preferred_element_type=jnp.float32)
    acc_ref[GH:NG, :] += jnp.dot(s_ref[1], h,
                                 preferred_element_type=jnp.float32)

    @pl.when(k == pl.num_programs(0) - 1)
    def _():
        cnt = cnt_ref[...]
        pooled = (acc_ref[...] + cnt * b_ref[...]) / jnp.maximum(cnt, 1.0)
        m = jnp.max(pooled, axis=1, keepdims=True)
        shifted = pooled - m
        out_ref[...] = shifted - jnp.log(
            jnp.sum(jnp.exp(shifted), axis=1, keepdims=True))


def kernel(x, edge_index, edge_weights, batch, W, b):
    n, d = x.shape
    cdim = W.shape[1]

    batp = jnp.concatenate([batch, jnp.zeros((NP - n,), jnp.int32)])
    eif = edge_index.reshape(2 * E)
    sp, cnt = _gcn_sc(eif, edge_weights, batp)
    s3 = sp.reshape(2, GH, NP)
    cnt_col = cnt[0, :NG].reshape(NG, 1)
    b_row = b.reshape(1, cdim)

    h = pl.pallas_call(
        _h_tc_body,
        grid=(NP // _BN,),
        in_specs=[
            pl.BlockSpec((_BN, d), lambda k: (k, 0)),
            pl.BlockSpec((d, cdim), lambda k: (0, 0)),
        ],
        out_specs=pl.BlockSpec((_BN, cdim), lambda k: (k, 0)),
        out_shape=jax.ShapeDtypeStruct((NP, cdim), jnp.float32),
    )(x, W)

    out = pl.pallas_call(
        _p_tc_body,
        grid=(NP // _BN,),
        in_specs=[
            pl.BlockSpec((2, GH, _BN), lambda k: (0, 0, k)),
            pl.BlockSpec((_BN, cdim), lambda k: (k, 0)),
            pl.BlockSpec((NG, 1), lambda k: (0, 0)),
            pl.BlockSpec((1, cdim), lambda k: (0, 0)),
        ],
        out_specs=pl.BlockSpec((NG, cdim), lambda k: (0, 0)),
        out_shape=jax.ShapeDtypeStruct((NG, cdim), jnp.float32),
        scratch_shapes=[pltpu.VMEM((NG, cdim), jnp.float32)],
    )(s3, h, cnt_col, b_row)

    return out


# speedup vs baseline: 1.0361x; 1.0228x over previous
import functools

import jax
import jax.numpy as jnp
from jax import lax
from jax.experimental import pallas as pl
from jax.experimental.pallas import tpu as pltpu
from jax.experimental.pallas import tpu_sc as plsc

N = 10000
E = 160000
NG = 128
NP = 10240
ET = E // 16
FB = ET // 128
TAIL = ET - FB * 128
NPT = NP // 16
GH = NG // 2
DEAD = NP * GH
SW = NP * GH + 128
GPAD = 200

_MESH = plsc.VectorSubcoreMesh(core_axis_name="c", subcore_axis_name="s")


def _zero_fill(ref, nwords):
    z = jnp.zeros((16,), ref.dtype)

    def body(i, carry):
        ref[pl.ds(i * 16, 16)] = z
        return carry

    lax.fori_loop(0, nwords // 16, body, 0)


def _newton_rsqrt(d):
    i = plsc.bitcast(d, jnp.int32)
    i = 0x5F3759DF - lax.shift_right_logical(i, 1)
    y = plsc.bitcast(i, jnp.float32)
    for _ in range(3):
        y = y * (1.5 - 0.5 * d * y * y)
    return y


@functools.partial(
    pl.kernel,
    out_type=(
        jax.ShapeDtypeStruct((2, NP * GH), jnp.float32),
        jax.ShapeDtypeStruct((2, 256), jnp.float32),
    ),
    mesh=_MESH,
    compiler_params=pltpu.CompilerParams(needs_layout_passes=False),
    scratch_types=[
        pltpu.VMEM((ET,), jnp.int32),
        pltpu.VMEM((ET,), jnp.int32),
        pltpu.VMEM((ET,), jnp.float32),
        pltpu.VMEM((NP,), jnp.int32),
        pltpu.VMEM((NP,), jnp.float32),
        pltpu.VMEM((NP,), jnp.float32),
        pltpu.VMEM((NPT,), jnp.float32),
        pltpu.VMEM((NPT,), jnp.float32),
        pltpu.VMEM((NPT,), jnp.float32),
        pltpu.VMEM((128,), jnp.int32),
        pltpu.VMEM((128,), jnp.int32),
        pltpu.VMEM((128,), jnp.int32),
        pltpu.VMEM((128,), jnp.int32),
        pltpu.VMEM((128,), jnp.int32),
        pltpu.VMEM((128,), jnp.int32),
        pltpu.VMEM((128,), jnp.float32),
        pltpu.VMEM((128,), jnp.float32),
        pltpu.VMEM((128,), jnp.float32),
        pltpu.VMEM((128,), jnp.float32),
        pltpu.VMEM((128,), jnp.float32),
        pltpu.VMEM((128,), jnp.float32),
        pltpu.VMEM((16,), jnp.int32),
        pltpu.VMEM((16,), jnp.float32),
        pltpu.VMEM((8192,), jnp.float32),
        pltpu.VMEM_SHARED((NP,), jnp.float32),
        pltpu.VMEM_SHARED((256,), jnp.float32),
        pltpu.VMEM_SHARED((NP,), jnp.float32),
        pltpu.VMEM_SHARED((NP,), jnp.float32),
        pltpu.VMEM_SHARED((SW,), jnp.float32),
        pltpu.SemaphoreType.DMA,
        pltpu.SemaphoreType.DMA,
        pltpu.SemaphoreType.DMA,
        pltpu.SemaphoreType.DMA,
        pltpu.SemaphoreType.DMA,
        pltpu.SemaphoreType.DMA,
        pltpu.SemaphoreType.DMA,
        pltpu.SemaphoreType.DMA,
    ],
)
def _gcn_sc(eif, ew, bat, sp, cnt,
            srcv, dstv, ewv, batv, disv, invv, degv, disl, invl,
            idx0, idx1, idx2, idx3, idx4, idx5,
            val0, val1, val2, val3, val4, val5, idxt, valt, zb,
            degsh, cntsh, dissh, invsh, ssh,
            sem0, sem1, sem2, sem3, sem4, sem5, sem6, sem7):
    c = lax.axis_index("c")
    s = lax.axis_index("s")
    glo = c * GH
    lanes = lax.iota(jnp.int32, 16)
    slots = ((idx0, val0, sem0), (idx1, val1, sem1),
             (idx2, val2, sem2), (idx3, val3, sem3),
             (idx4, val4, sem6), (idx5, val5, sem7))

    pltpu.async_copy(eif.at[pl.ds(s * ET, ET)], srcv, sem4)
    pltpu.async_copy(eif.at[pl.ds(E + s * ET, ET)], dstv, sem5)
    pltpu.async_copy(ew.at[pl.ds(s * ET, ET)], ewv, sem6)
    pltpu.async_copy(bat, batv.at[pl.ds(0, N)], sem7)
    for t in range((NP - N) // 16):
        batv[pl.ds(N + t * 16, 16)] = jnp.zeros((16,), jnp.int32)

    _zero_fill(zb, 8192)

    ts = NP * GH // 16

    for i in range(4):
        pltpu.async_copy(zb, ssh.at[pl.ds(s * ts + i * 8192, 8192)],
                         (sem0, sem1, sem2, sem3)[i])
    pltpu.sync_copy(zb.at[pl.ds(0, NPT)], degsh.at[pl.ds(s * NPT, NPT)])

    @pl.when(s == 0)
    def _():
        pltpu.sync_copy(zb.at[pl.ds(0, 256)], cntsh)

    pltpu.make_async_copy(zb, ssh.at[pl.ds(s * ts, 8192)], sem0).wait()
    pltpu.sync_copy(zb, ssh.at[pl.ds(s * ts + 4 * 8192, 8192)])
    for i in range(1, 4):
        pltpu.make_async_copy(zb, ssh.at[pl.ds(s * ts + i * 8192, 8192)],
                              (sem0, sem1, sem2, sem3)[i]).wait()
    pltpu.make_async_copy(eif.at[pl.ds(s * ET, ET)], srcv, sem4).wait()
    pltpu.make_async_copy(eif.at[pl.ds(E + s * ET, ET)], dstv, sem5).wait()
    pltpu.make_async_copy(ew.at[pl.ds(s * ET, ET)], ewv, sem6).wait()
    pltpu.make_async_copy(bat, batv.at[pl.ds(0, N)], sem7).wait()

    def _pipeline(nbatches, make_batch, target):
        nslots = min(len(slots), nbatches)

        def fire(j, sl):
            idxr, valr, sem = slots[sl]
            make_batch(j, idxr, valr)
            pltpu.async_copy(valr, target.at[idxr], sem, add=True)

        def drain(sl):
            idxr, valr, sem = slots[sl]
            pltpu.make_async_copy(valr, target.at[idxr], sem).wait()

        for b in range(nslots):
            fire(b, b)

        def grp(g, carry):
            for b in range(nslots):
                drain(b)
                fire(nslots + g * nslots + b, b)
            return carry

        ngroups = (nbatches - nslots) // nslots
        lax.fori_loop(0, ngroups, grp, 0)
        rem = (nbatches - nslots) % nslots
        for r in range(rem):
            drain(r)
            fire(nbatches - rem + r, r)
        for b in range(nslots):
            drain(b)

    def deg_batch(j, idxr, valr):
        def sub(k, carry):
            off = j * 128 + k * 16
            idxr[pl.ds(k * 16, 16)] = dstv[pl.ds(off, 16)]
            valr[pl.ds(k * 16, 16)] = ewv[pl.ds(off, 16)]
            return carry

        lax.fori_loop(0, 8, sub, 0)

    _pipeline(FB, deg_batch, degsh)

    idxt[...] = dstv[pl.ds(FB * 128, 16)]
    valt[...] = ewv[pl.ds(FB * 128, 16)]
    pltpu.sync_copy(valt, degsh.at[idxt], add=True)

    def count_batch(j, idxr, valr):
        def sub(k, carry):
            base = j * 128 + k * 16
            nglob = s * NPT + base + lanes
            b16 = batv[pl.ds(s * NPT + base, 16)]
            valid = nglob < N
            idxr[pl.ds(k * 16, 16)] = jnp.where(valid, b16, GPAD + lanes)
            valr[pl.ds(k * 16, 16)] = jnp.where(valid, 1.0, 0.0)
            return carry

        lax.fori_loop(0, 8, sub, 0)

    _pipeline(NPT // 128, count_batch, cntsh)

    plsc.subcore_barrier()

    pltpu.sync_copy(degsh.at[pl.ds(s * NPT, NPT)], degv)

    def newton(i, carry):
        d = degv[pl.ds(i * 16, 16)] + 1.0
        y = _newton_rsqrt(d)
        disl[pl.ds(i * 16, 16)] = y
        invl[pl.ds(i * 16, 16)] = y * y
        return carry

    lax.fori_loop(0, NPT // 16, newton, 0)
    pltpu.sync_copy(disl, dissh.at[pl.ds(s * NPT, NPT)])
    pltpu.sync_copy(invl, invsh.at[pl.ds(s * NPT, NPT)])
    plsc.subcore_barrier()
    pltpu.sync_copy(dissh, disv)
    pltpu.sync_copy(invsh, invv)

    def s_batch(j, idxr, valr):
        def sub(k, carry):
            off = j * 128 + k * 16
            s16 = srcv[pl.ds(off, 16)]
            d16 = dstv[pl.ds(off, 16)]
            w16 = ewv[pl.ds(off, 16)]
            dd = plsc.load_gather(disv, [d16])
            ds_ = plsc.load_gather(disv, [s16])
            gi = plsc.load_gather(batv, [d16]) - glo
            own = (gi >= 0) & (gi < GH)
            dead = DEAD + ((lanes + k * 16) & 127)
            idxr[pl.ds(k * 16, 16)] = jnp.where(own, gi * NP + s16, dead)
            valr[pl.ds(k * 16, 16)] = jnp.where(own, dd * ds_ * w16, 0.0)
            return carry

        lax.fori_loop(0, 8, sub, 0)

    _pipeline(FB, s_batch, ssh)

    s16 = srcv[pl.ds(FB * 128, 16)]
    d16 = dstv[pl.ds(FB * 128, 16)]
    w16 = ewv[pl.ds(FB * 128, 16)]
    dd = plsc.load_gather(disv, [d16])
    ds_ = plsc.load_gather(disv, [s16])
    gi = plsc.load_gather(batv, [d16]) - glo
    own = (gi >= 0) & (gi < GH)
    idxt[...] = jnp.where(own, gi * NP + s16, DEAD + lanes)
    valt[...] = jnp.where(own, dd * ds_ * w16, 0.0)
    pltpu.sync_copy(valt, ssh.at[idxt], add=True)

    def self_batch(j, idxr, valr):
        def sub(k, carry):
            base = j * 128 + k * 16
            n16 = s * NPT + base + lanes
            gi = plsc.load_gather(batv, [n16]) - glo
            v16 = plsc.load_gather(invv, [n16])
            own = (gi >= 0) & (gi < GH) & (n16 < N)
            dead = DEAD + ((lanes + k * 16) & 127)
            idxr[pl.ds(k * 16, 16)] = jnp.where(own, gi * NP + n16, dead)
            valr[pl.ds(k * 16, 16)] = jnp.where(own, v16, 0.0)
            return carry

        lax.fori_loop(0, 8, sub, 0)

    _pipeline(NPT // 128, self_batch, ssh)

    plsc.subcore_barrier()

    pltpu.sync_copy(ssh.at[pl.ds(s * ts, ts)], sp.at[c, pl.ds(s * ts, ts)])

    @pl.when(s == 0)
    def _():
        pltpu.sync_copy(cntsh, cnt.at[c])


_BN = 1024


def _h_tc_body(x_ref, w_ref, h_ref):
    k = pl.program_id(0)
    h = jnp.dot(x_ref[...], w_ref[...], preferred_element_type=jnp.float32)
    rid = lax.broadcasted_iota(jnp.int32, h.shape, 0)
    h_ref[...] = jnp.where(rid < N - k * _BN, h, 0.0)


def _p_tc_body(s_ref, h_ref, cnt_ref, b_ref, out_ref, acc_ref):
    k = pl.program_id(0)

    @pl.when(k == 0)
    def _():
        acc_ref[...] = jnp.zeros_like(acc_ref)

    h = h_ref[...]
    acc_ref[0:GH, :] += jnp.dot(s_ref[0], h,
                                preferred_element_type=jnp.float32)
    acc_ref[GH:NG, :] += jnp.dot(s_ref[1], h,
                                 preferred_element_type=jnp.float32)

    @pl.when(k == pl.num_programs(0) - 1)
    def _():
        cnt = cnt_ref[...]
        pooled = (acc_ref[...] + cnt * b_ref[...]) / jnp.maximum(cnt, 1.0)
        m = jnp.max(pooled, axis=1, keepdims=True)
        shifted = pooled - m
        out_ref[...] = shifted - jnp.log(
            jnp.sum(jnp.exp(shifted), axis=1, keepdims=True))


def kernel(x, edge_index, edge_weights, batch, W, b):
    n, d = x.shape
    cdim = W.shape[1]

    eif = edge_index.reshape(2 * E)
    sp, cnt = _gcn_sc(eif, edge_weights, batch)
    s3 = sp.reshape(2, GH, NP)
    cnt_col = cnt[0, :NG].reshape(NG, 1)
    b_row = b.reshape(1, cdim)

    h = pl.pallas_call(
        _h_tc_body,
        grid=(NP // _BN,),
        in_specs=[
            pl.BlockSpec((_BN, d), lambda k: (k, 0)),
            pl.BlockSpec((d, cdim), lambda k: (0, 0)),
        ],
        out_specs=pl.BlockSpec((_BN, cdim), lambda k: (k, 0)),
        out_shape=jax.ShapeDtypeStruct((NP, cdim), jnp.float32),
    )(x, W)

    out = pl.pallas_call(
        _p_tc_body,
        grid=(NP // _BN,),
        in_specs=[
            pl.BlockSpec((2, GH, _BN), lambda k: (0, 0, k)),
            pl.BlockSpec((_BN, cdim), lambda k: (k, 0)),
            pl.BlockSpec((NG, 1), lambda k: (0, 0)),
            pl.BlockSpec((1, cdim), lambda k: (0, 0)),
        ],
        out_specs=pl.BlockSpec((NG, cdim), lambda k: (0, 0)),
        out_shape=jax.ShapeDtypeStruct((NG, cdim), jnp.float32),
        scratch_shapes=[pltpu.VMEM((NG, cdim), jnp.float32)],
    )(s3, h, cnt_col, b_row)

    return out
